# Initial kernel scaffold; baseline (speedup 1.0000x reference)
import jax
import jax.numpy as jnp
from jax.experimental import pallas as pl


def kernel(z, W1, b1, W2, b2, temperature):
    raise NotImplementedError("write your pallas kernel here")


# speedup vs baseline: 1.9222x; 1.9222x over previous
import jax
import jax.numpy as jnp
from jax.experimental import pallas as pl
from jax.experimental.pallas import tpu as pltpu

_NB = 8
_BM = 1024
_BN = 512


def _router_block(temp_ref, z_ref, w1_ref, b1_ref, w2_ref, b2_ref,
                  weights_ref, idx_ref, acc_ref):
    j = pl.program_id(1)
    nj = pl.num_programs(1)

    h = jax.lax.dot_general(
        z_ref[...], w1_ref[...], (((1,), (1,)), ((), ())),
        preferred_element_type=jnp.float32)
    h = h + b1_ref[...]
    h = 0.5 * h * (1.0 + jax.lax.erf(h * 0.7071067811865476))
    part = jax.lax.dot_general(
        h, w2_ref[...], (((1,), (1,)), ((), ())),
        preferred_element_type=jnp.float32)

    @pl.when(j == 0)
    def _init():
        acc_ref[...] = part

    @pl.when(j > 0)
    def _accum():
        acc_ref[...] += part

    @pl.when(j == nj - 1)
    def _epilogue():
        logits = acc_ref[...] + b2_ref[...]
        iota = jax.lax.broadcasted_iota(jnp.int32, logits.shape, 1)
        m1 = jnp.max(logits, axis=1, keepdims=True)
        idx1 = jnp.min(jnp.where(logits == m1, iota, _NB), axis=1,
                       keepdims=True)
        masked = jnp.where(iota == idx1, -jnp.inf, logits)
        m2 = jnp.max(masked, axis=1, keepdims=True)
        idx2 = jnp.min(jnp.where(masked == m2, iota, _NB), axis=1,
                       keepdims=True)
        e = jnp.exp((m2 - m1) * temp_ref[0])
        w_hi = 1.0 / (1.0 + e)
        w_lo = e / (1.0 + e)
        weights_ref[...] = jnp.where(
            iota == idx1, w_hi, jnp.where(iota == idx2, w_lo, 0.0))
        pair = jax.lax.broadcasted_iota(jnp.int32, idx_ref.shape, 1)
        idx_ref[...] = jnp.where(pair == 0, idx1, idx2)


@jax.jit
def kernel(z, W1, b1, W2, b2, temperature):
    n, d = z.shape
    inv_temp = 1.0 / (jax.nn.softplus(temperature) + 0.1)
    inv_temp = jnp.reshape(inv_temp, (1,)).astype(jnp.float32)
    b1r = jnp.reshape(b1, (1, d))
    b2r = jnp.reshape(b2, (1, _NB))
    grid = (n // _BM, d // _BN)
    weights, idx = pl.pallas_call(
        _router_block,
        grid=grid,
        in_specs=[
            pl.BlockSpec(memory_space=pltpu.SMEM),
            pl.BlockSpec((_BM, d), lambda i, j: (i, 0)),
            pl.BlockSpec((_BN, d), lambda i, j: (j, 0)),
            pl.BlockSpec((1, _BN), lambda i, j: (0, j)),
            pl.BlockSpec((_NB, _BN), lambda i, j: (0, j)),
            pl.BlockSpec((1, _NB), lambda i, j: (0, 0)),
        ],
        out_specs=[
            pl.BlockSpec((_BM, _NB), lambda i, j: (i, 0)),
            pl.BlockSpec((_BM, 2), lambda i, j: (i, 0)),
        ],
        out_shape=[
            jax.ShapeDtypeStruct((n, _NB), jnp.float32),
            jax.ShapeDtypeStruct((n, 2), jnp.int32),
        ],
        scratch_shapes=[pltpu.VMEM((_BM, _NB), jnp.float32)],
        compiler_params=pltpu.CompilerParams(
            dimension_semantics=("parallel", "arbitrary")),
    )(inv_temp, z, W1, b1r, W2, b2r)
    return weights, idx
